# Initial kernel scaffold; baseline (speedup 1.0000x reference)
import jax
import jax.numpy as jnp
from jax.experimental import pallas as pl


def kernel(x, edge_index, W_in, b_in, W1_l, b1_l, W1_r, gn1_w, gn1_b, gn1_a, W2_l, b2_l, W2_r, gn2_w, gn2_b, gn2_a, W3_l, b3_l, W3_r, gn3_w, gn3_b, gn3_a, W_out, b_out):
    raise NotImplementedError("write your pallas kernel here")


# speedup vs baseline: 4.5151x; 4.5151x over previous
import jax
import jax.numpy as jnp
from jax import lax
from jax.experimental import pallas as pl
from jax.experimental.pallas import tpu as pltpu
from jax.experimental.pallas import tpu_sc as plsc

N = 10000
E = 320000
F = 128
NC = 2
NS = 16
NW = NC * NS
EPW = E // NW
WIN = 80
NWIN = EPW // WIN
RPS = 624
RTAIL = N - RPS * NS


def _sc_mesh():
    return plsc.VectorSubcoreMesh(core_axis_name="c", subcore_axis_name="s",
                                  num_cores=NC, num_subcores=NS)


def _rows_copy(sid, src, dst):
    r0 = sid * RPS
    pltpu.sync_copy(src.at[pl.ds(r0, RPS)], dst.at[pl.ds(r0, RPS)])

    @pl.when(sid == 0)
    def _():
        pltpu.sync_copy(src.at[pl.ds(RPS * NS, RTAIL)],
                        dst.at[pl.ds(RPS * NS, RTAIL)])


def _sc_agg_body(h_hbm, src_hbm, dst_hbm, zf_hbm, acc_out,
                 src_v, dst_v, rows_v, acc_sh, sem):
    cid = lax.axis_index("c")
    sid = lax.axis_index("s")

    _rows_copy(sid, zf_hbm, acc_sh)
    plsc.subcore_barrier()

    base0 = (cid * NS + sid) * EPW

    @pl.loop(0, NWIN)
    def _(j):
        base = base0 + j * WIN
        pltpu.sync_copy(src_hbm.at[pl.ds(base, WIN)], src_v)
        pltpu.sync_copy(dst_hbm.at[pl.ds(base, WIN)], dst_v)
        pltpu.async_copy(h_hbm.at[src_v], rows_v, sem).wait()
        pltpu.sync_copy(rows_v, acc_sh.at[dst_v], add=True)

    plsc.subcore_barrier()
    _rows_copy(sid, acc_sh, acc_out.at[cid])


def _make_sc_agg():
    return pl.kernel(
        _sc_agg_body,
        out_type=jax.ShapeDtypeStruct((NC, N, F), jnp.float32),
        mesh=_sc_mesh(),
        scratch_types=[pltpu.VMEM((WIN,), jnp.int32),
                       pltpu.VMEM((WIN,), jnp.int32),
                       pltpu.VMEM((WIN, F), jnp.float32),
                       pltpu.VMEM_SHARED((N, F), jnp.float32),
                       pltpu.SemaphoreType.DMA])


def _sc_cnt_body(dst_hbm, zf_hbm, ones_hbm, cnt_out,
                 dst_v, ones_v, cnt_sh):
    cid = lax.axis_index("c")
    sid = lax.axis_index("s")

    _rows_copy(sid, zf_hbm, cnt_sh)
    pltpu.sync_copy(ones_hbm, ones_v)
    plsc.subcore_barrier()

    base0 = (cid * NS + sid) * EPW

    @pl.loop(0, NWIN)
    def _(j):
        base = base0 + j * WIN
        pltpu.sync_copy(dst_hbm.at[pl.ds(base, WIN)], dst_v)
        pltpu.sync_copy(ones_v, cnt_sh.at[dst_v], add=True)

    plsc.subcore_barrier()
    _rows_copy(sid, cnt_sh, cnt_out.at[cid])


def _make_sc_cnt():
    return pl.kernel(
        _sc_cnt_body,
        out_type=jax.ShapeDtypeStruct((NC, N, F), jnp.float32),
        mesh=_sc_mesh(),
        scratch_types=[pltpu.VMEM((WIN,), jnp.int32),
                       pltpu.VMEM((WIN, F), jnp.float32),
                       pltpu.VMEM_SHARED((N, F), jnp.float32)])


_RB = 1000
_GRID = N // _RB


def _dot(a, b):
    return jnp.dot(a, b, preferred_element_type=jnp.float32,
                   precision=lax.Precision.HIGHEST)


def _in_proj_body(x_ref, w_ref, b_ref, o_ref):
    o_ref[...] = _dot(x_ref[...], w_ref[...]) + b_ref[...]


def _in_proj(x, w, b):
    return pl.pallas_call(
        _in_proj_body,
        grid=(_GRID,),
        in_specs=[pl.BlockSpec((_RB, F), lambda i: (i, 0)),
                  pl.BlockSpec((F, F), lambda i: (0, 0)),
                  pl.BlockSpec((1, F), lambda i: (0, 0))],
        out_specs=pl.BlockSpec((_RB, F), lambda i: (i, 0)),
        out_shape=jax.ShapeDtypeStruct((N, F), jnp.float32),
    )(x, w, b.reshape(1, F))


def _mix_body(p_ref, c_ref, h_ref, wl_ref, wr_ref, b_ref, y_ref, s_ref):
    cnt = c_ref[0, :, 0:1] + c_ref[1, :, 0:1]
    inv = 1.0 / jnp.maximum(cnt, 1.0)
    agg = (p_ref[0] + p_ref[1]) * inv
    y = _dot(agg, wl_ref[...]) + _dot(h_ref[...], wr_ref[...]) + b_ref[...]
    y_ref[...] = y
    stats = jnp.concatenate([jnp.sum(y, axis=0, keepdims=True),
                             jnp.sum(y * y, axis=0, keepdims=True)], axis=0)
    i = pl.program_id(0)

    @pl.when(i == 0)
    def _():
        s_ref[...] = stats

    @pl.when(i > 0)
    def _():
        s_ref[...] += stats


def _mix(parts, cnts, h, wl, wr, b):
    return pl.pallas_call(
        _mix_body,
        grid=(_GRID,),
        in_specs=[pl.BlockSpec((NC, _RB, F), lambda i: (0, i, 0)),
                  pl.BlockSpec((NC, _RB, F), lambda i: (0, i, 0)),
                  pl.BlockSpec((_RB, F), lambda i: (i, 0)),
                  pl.BlockSpec((F, F), lambda i: (0, 0)),
                  pl.BlockSpec((F, F), lambda i: (0, 0)),
                  pl.BlockSpec((1, F), lambda i: (0, 0))],
        out_specs=[pl.BlockSpec((_RB, F), lambda i: (i, 0)),
                   pl.BlockSpec((2, F), lambda i: (0, 0))],
        out_shape=[jax.ShapeDtypeStruct((N, F), jnp.float32),
                   jax.ShapeDtypeStruct((2, F), jnp.float32)],
    )(parts, cnts, h, wl, wr, b.reshape(1, F))


def _normed(y, s_ref, w_ref, b_ref, a_ref):
    mean = s_ref[0:1, :] * (1.0 / N)
    msq = s_ref[1:2, :] * (1.0 / N)
    a = a_ref[...]
    var = msq - mean * mean * (2.0 * a - a * a)
    xc = y - a * mean
    t = w_ref[...] * xc / jnp.sqrt(var + 1e-5) + b_ref[...]
    return jnp.maximum(t, 0.1 * t)


def _norm_body(y_ref, s_ref, w_ref, b_ref, a_ref, o_ref):
    o_ref[...] = _normed(y_ref[...], s_ref, w_ref, b_ref, a_ref)


def _norm(y, s, w, b, a):
    return pl.pallas_call(
        _norm_body,
        grid=(_GRID,),
        in_specs=[pl.BlockSpec((_RB, F), lambda i: (i, 0)),
                  pl.BlockSpec((2, F), lambda i: (0, 0)),
                  pl.BlockSpec((1, F), lambda i: (0, 0)),
                  pl.BlockSpec((1, F), lambda i: (0, 0)),
                  pl.BlockSpec((1, F), lambda i: (0, 0))],
        out_specs=pl.BlockSpec((_RB, F), lambda i: (i, 0)),
        out_shape=jax.ShapeDtypeStruct((N, F), jnp.float32),
    )(y, s, w.reshape(1, F), b.reshape(1, F), a.reshape(1, F))


def _norm_out_body(y_ref, s_ref, w_ref, b_ref, a_ref, wo_ref, bo_ref, o_ref):
    t = _normed(y_ref[...], s_ref, w_ref, b_ref, a_ref)
    o_ref[...] = _dot(t, wo_ref[...]) + bo_ref[...]


def _norm_out(y, s, w, b, a, wo, bo):
    return pl.pallas_call(
        _norm_out_body,
        grid=(_GRID,),
        in_specs=[pl.BlockSpec((_RB, F), lambda i: (i, 0)),
                  pl.BlockSpec((2, F), lambda i: (0, 0)),
                  pl.BlockSpec((1, F), lambda i: (0, 0)),
                  pl.BlockSpec((1, F), lambda i: (0, 0)),
                  pl.BlockSpec((1, F), lambda i: (0, 0)),
                  pl.BlockSpec((F, F), lambda i: (0, 0)),
                  pl.BlockSpec((1, F), lambda i: (0, 0))],
        out_specs=pl.BlockSpec((_RB, F), lambda i: (i, 0)),
        out_shape=jax.ShapeDtypeStruct((N, F), jnp.float32),
    )(y, s, w.reshape(1, F), b.reshape(1, F), a.reshape(1, F),
      wo, bo.reshape(1, F))


def kernel(x, edge_index, W_in, b_in,
           W1_l, b1_l, W1_r, gn1_w, gn1_b, gn1_a,
           W2_l, b2_l, W2_r, gn2_w, gn2_b, gn2_a,
           W3_l, b3_l, W3_r, gn3_w, gn3_b, gn3_a,
           W_out, b_out):
    src = edge_index[0]
    dst = edge_index[1]
    zf = jnp.zeros((N, F), jnp.float32)
    ones = jnp.ones((WIN, F), jnp.float32)

    _sc_agg = _make_sc_agg()
    _sc_cnt = _make_sc_cnt()

    cnt = _sc_cnt(dst, zf, ones)
    h0 = _in_proj(x, W_in, b_in)
    p1 = _sc_agg(h0, src, dst, zf)
    y1, s1 = _mix(p1, cnt, h0, W1_l, W1_r, b1_l)
    h1 = _norm(y1, s1, gn1_w, gn1_b, gn1_a)
    p2 = _sc_agg(h1, src, dst, zf)
    y2, s2 = _mix(p2, cnt, h1, W2_l, W2_r, b2_l)
    h2 = _norm(y2, s2, gn2_w, gn2_b, gn2_a)
    p3 = _sc_agg(h2, src, dst, zf)
    y3, s3 = _mix(p3, cnt, h2, W3_l, W3_r, b3_l)
    return _norm_out(y3, s3, gn3_w, gn3_b, gn3_a, W_out, b_out)
